# Initial kernel scaffold; baseline (speedup 1.0000x reference)
import jax
import jax.numpy as jnp
from jax.experimental import pallas as pl


def kernel(edge_diff, edge_idx, n_atoms):
    raise NotImplementedError("write your pallas kernel here")


# speedup vs baseline: 2.0844x; 2.0844x over previous
import jax
import jax.numpy as jnp
from jax import lax
from jax.experimental import pallas as pl
from jax.experimental.pallas import tpu as pltpu
from jax.experimental.pallas import tpu_sc as plsc

E = 6_400_000
N = 100_000
W = 3 * N
NC = 2
NS = 16
L = 16
NW = NC * NS
CHUNK = 2048
CW = CHUNK * 3
GROUPS = CHUNK // L
TOTAL_CHUNKS = E // CHUNK
BASE_CHUNKS = TOTAL_CHUNKS // NW
EXTRA = TOTAL_CHUNKS % NW
WPS = 18752
WPS_LAST = W - (NS - 1) * WPS


def _sc_body(diff_hbm, idx_hbm, zeros_hbm, out_hbm,
             diff_v, idx_v, pos_v, neg_v, iiw_v, jjw_v, acc_s, sem_sc):
    cid = lax.axis_index("c")
    sid = lax.axis_index("s")
    wid = cid * NS + sid

    r0 = sid * WPS

    @pl.when(sid < NS - 1)
    def _():
        pltpu.sync_copy(zeros_hbm.at[pl.ds(r0, WPS)], acc_s.at[pl.ds(r0, WPS)])

    @pl.when(sid == NS - 1)
    def _():
        pltpu.sync_copy(zeros_hbm.at[pl.ds((NS - 1) * WPS, WPS_LAST)],
                        acc_s.at[pl.ds((NS - 1) * WPS, WPS_LAST)])

    plsc.subcore_barrier()

    start = wid * BASE_CHUNKS + jnp.minimum(wid, EXTRA)
    nchunks = BASE_CHUNKS + (wid < EXTRA).astype(jnp.int32)

    iota = lax.iota(jnp.int32, L)
    magic = jnp.full((L,), 0x5F3759DF, jnp.int32)

    def do_chunk(ci, carry):
        e0 = (start + ci) * CHUNK
        pltpu.sync_copy(diff_hbm.at[pl.ds(e0 * 3, CW)], diff_v)
        pltpu.sync_copy(idx_hbm.at[pl.ds(e0 * 2, CHUNK * 2)], idx_v)

        def do_group(g, c_):
            rows = g * L + iota
            p0 = rows * 3
            p1 = p0 + 1
            p2 = p0 + 2
            q0 = rows * 2
            dx = plsc.load_gather(diff_v, [p0])
            dy = plsc.load_gather(diff_v, [p1])
            dz = plsc.load_gather(diff_v, [p2])
            ii = plsc.load_gather(idx_v, [q0])
            jj = plsc.load_gather(idx_v, [q0 + 1])
            r2 = dx * dx + dy * dy + dz * dz + 1e-12
            bi = plsc.bitcast(r2, jnp.int32)
            y = plsc.bitcast(magic - lax.shift_right_logical(bi, 1), jnp.float32)
            xh = r2 * 0.5
            y = y * (1.5 - xh * y * y)
            y = y * (1.5 - xh * y * y)
            y = y * (1.5 - xh * y * y)
            s = 1.0 - y
            t = y - 1.0
            plsc.store_scatter(pos_v, [p0], s * dx)
            plsc.store_scatter(pos_v, [p1], s * dy)
            plsc.store_scatter(pos_v, [p2], s * dz)
            plsc.store_scatter(neg_v, [p0], t * dx)
            plsc.store_scatter(neg_v, [p1], t * dy)
            plsc.store_scatter(neg_v, [p2], t * dz)
            wa = ii * 3
            wb = jj * 3
            plsc.store_scatter(iiw_v, [p0], wa)
            plsc.store_scatter(iiw_v, [p1], wa + 1)
            plsc.store_scatter(iiw_v, [p2], wa + 2)
            plsc.store_scatter(jjw_v, [p0], wb)
            plsc.store_scatter(jjw_v, [p1], wb + 1)
            plsc.store_scatter(jjw_v, [p2], wb + 2)
            return c_

        lax.fori_loop(0, GROUPS, do_group, 0, unroll=False)

        dpos = pltpu.async_copy(pos_v, acc_s.at[iiw_v], sem_sc, add=True)
        dneg = pltpu.async_copy(neg_v, acc_s.at[jjw_v], sem_sc, add=True)
        dpos.wait()
        dneg.wait()
        return carry

    lax.fori_loop(0, nchunks, do_chunk, 0, unroll=False)

    plsc.subcore_barrier()

    @pl.when(sid < NS - 1)
    def _():
        pltpu.sync_copy(acc_s.at[pl.ds(r0, WPS)],
                        out_hbm.at[cid, pl.ds(r0, WPS)])

    @pl.when(sid == NS - 1)
    def _():
        pltpu.sync_copy(acc_s.at[pl.ds((NS - 1) * WPS, WPS_LAST)],
                        out_hbm.at[cid, pl.ds((NS - 1) * WPS, WPS_LAST)])


def _combine_body(a_ref, b_ref, o_ref):
    o_ref[...] = a_ref[...] + b_ref[...]


def kernel(edge_diff, edge_idx, n_atoms):
    del n_atoms
    zeros = jnp.zeros((W,), jnp.float32)
    mesh = plsc.VectorSubcoreMesh(core_axis_name="c", subcore_axis_name="s")
    partials = pl.kernel(
        _sc_body,
        out_type=jax.ShapeDtypeStruct((NC, W), jnp.float32),
        compiler_params=pltpu.CompilerParams(
            needs_layout_passes=False, use_tc_tiling_on_sc=False),
        mesh=mesh,
        scratch_types=[
            pltpu.VMEM((CW,), jnp.float32),
            pltpu.VMEM((CHUNK * 2,), jnp.int32),
            pltpu.VMEM((CW,), jnp.float32),
            pltpu.VMEM((CW,), jnp.float32),
            pltpu.VMEM((CW,), jnp.int32),
            pltpu.VMEM((CW,), jnp.int32),
            pltpu.VMEM_SHARED((W,), jnp.float32),
            pltpu.SemaphoreType.DMA,
        ],
    )(edge_diff.reshape(E * 3), edge_idx.reshape(E * 2), zeros)

    pa = partials[0].reshape(300, 1000)
    pb = partials[1].reshape(300, 1000)
    out = pl.pallas_call(
        _combine_body,
        out_shape=jax.ShapeDtypeStruct((300, 1000), jnp.float32),
    )(pa, pb)
    return out.reshape(N, 3)
